# Initial kernel scaffold; baseline (speedup 1.0000x reference)
import jax
import jax.numpy as jnp
from jax import lax
from jax.experimental import pallas as pl
from jax.experimental.pallas import tpu as pltpu
from jax.experimental.pallas import tpu_sc as plsc

N_USERS = 5000
N_ITEMS = 5000
NN = N_USERS + N_ITEMS
NP = 10240
EMB = 128
NLAYERS = 3
E = 320000
NCORES = 2
NSUB = 16
NTILES = NCORES * NSUB
CHUNK = 128
EPT = 10112
EP = EPT * NTILES
NCHUNKS = EPT // CHUNK
ROWS_PER_TILE = NP // NSUB
TRASH = NN
ALPHA = 1.0 / (NLAYERS + 1)


def _sc_layer_body(cur_hbm, src_hbm, dst_hbm, w_hbm, zero_hbm, out_hbm,
                   sidx, didx, wbuf, rows, acc_sh, gsem):
    c = lax.axis_index("c")
    s = lax.axis_index("s")
    wid = s * NCORES + c
    r0 = s * ROWS_PER_TILE
    pltpu.sync_copy(zero_hbm.at[pl.ds(r0, ROWS_PER_TILE)],
                    acc_sh.at[pl.ds(r0, ROWS_PER_TILE)])
    plsc.subcore_barrier()

    ebase = wid * EPT

    def chunk(g, carry):
        e0 = ebase + g * CHUNK
        pltpu.sync_copy(src_hbm.at[pl.ds(e0, CHUNK)], sidx)
        pltpu.sync_copy(dst_hbm.at[pl.ds(e0, CHUNK)], didx)
        pltpu.sync_copy(w_hbm.at[pl.ds(e0, CHUNK)], wbuf)
        pltpu.async_copy(cur_hbm.at[sidx], rows, gsem).wait()

        def escale(e, _):
            w = wbuf[e]
            for cc in range(EMB // 16):
                sl = pl.ds(cc * 16, 16)
                rows[e, sl] = rows[e, sl] * w
            return 0

        lax.fori_loop(0, CHUNK, escale, 0)
        pltpu.sync_copy(rows, acc_sh.at[didx], add=True)
        return carry

    lax.fori_loop(0, NCHUNKS, chunk, 0)
    plsc.subcore_barrier()
    pltpu.sync_copy(acc_sh.at[pl.ds(r0, ROWS_PER_TILE)],
                    out_hbm.at[c, pl.ds(r0, ROWS_PER_TILE)])


_sc_layer = pl.kernel(
    _sc_layer_body,
    out_type=jax.ShapeDtypeStruct((NCORES, NP, EMB), jnp.float32),
    mesh=plsc.VectorSubcoreMesh(core_axis_name="c", subcore_axis_name="s",
                                num_cores=NCORES, num_subcores=NSUB),
    scratch_types=[
        pltpu.VMEM((CHUNK,), jnp.int32),
        pltpu.VMEM((CHUNK,), jnp.int32),
        pltpu.VMEM((CHUNK,), jnp.float32),
        pltpu.VMEM((CHUNK, EMB), jnp.float32),
        pltpu.VMEM_SHARED((NP, EMB), jnp.float32),
        pltpu.SemaphoreType.DMA,
    ],
)

_BLK = 1280


def _combine_mid_body(p_ref, acc_ref, cur_out, acc_out):
    cur = p_ref[0] + p_ref[1]
    cur_out[...] = cur
    acc_out[...] = acc_ref[...] + cur


def _combine_mid(parts, acc):
    return pl.pallas_call(
        _combine_mid_body,
        grid=(NP // _BLK,),
        in_specs=[pl.BlockSpec((NCORES, _BLK, EMB), lambda i: (0, i, 0)),
                  pl.BlockSpec((_BLK, EMB), lambda i: (i, 0))],
        out_specs=[pl.BlockSpec((_BLK, EMB), lambda i: (i, 0)),
                   pl.BlockSpec((_BLK, EMB), lambda i: (i, 0))],
        out_shape=[jax.ShapeDtypeStruct((NP, EMB), jnp.float32),
                   jax.ShapeDtypeStruct((NP, EMB), jnp.float32)],
    )(parts, acc)


def _combine_last_body(p_ref, acc_ref, out_ref):
    out_ref[...] = ALPHA * (acc_ref[...] + p_ref[0] + p_ref[1])


def _combine_last(parts, acc):
    return pl.pallas_call(
        _combine_last_body,
        grid=(NP // _BLK,),
        in_specs=[pl.BlockSpec((NCORES, _BLK, EMB), lambda i: (0, i, 0)),
                  pl.BlockSpec((_BLK, EMB), lambda i: (i, 0))],
        out_specs=pl.BlockSpec((_BLK, EMB), lambda i: (i, 0)),
        out_shape=jax.ShapeDtypeStruct((NP, EMB), jnp.float32),
    )(parts, acc)


def kernel(edge_index, edge_weight, user_emb, item_emb):
    src = edge_index[0].astype(jnp.int32)
    dst = edge_index[1].astype(jnp.int32)
    w = edge_weight.astype(jnp.float32)
    pad_e = EP - E
    src = jnp.concatenate([src, jnp.zeros((pad_e,), jnp.int32)])
    dst = jnp.concatenate([dst, jnp.full((pad_e,), TRASH, jnp.int32)])
    w = jnp.concatenate([w, jnp.zeros((pad_e,), jnp.float32)])
    ego = jnp.concatenate([user_emb, item_emb], axis=0)
    cur = jnp.pad(ego, ((0, NP - NN), (0, 0)))
    zeros = jnp.zeros((NP, EMB), jnp.float32)
    acc = cur
    out = None
    for layer in range(NLAYERS):
        parts = _sc_layer(cur, src, dst, w, zeros)
        if layer < NLAYERS - 1:
            cur, acc = _combine_mid(parts, acc)
        else:
            out = _combine_last(parts, acc)
    return (out[:N_USERS], out[N_USERS:NN])


# speedup vs baseline: 3.2311x; 3.2311x over previous
import jax
import jax.numpy as jnp
from jax import lax
from jax.experimental import pallas as pl
from jax.experimental.pallas import tpu as pltpu
from jax.experimental.pallas import tpu_sc as plsc

N_USERS = 5000
N_ITEMS = 5000
NN = N_USERS + N_ITEMS
NP = 10240
EMB = 128
NLAYERS = 3
E = 320000
NCORES = 2
NSUB = 16
NTILES = NCORES * NSUB
CHUNK = 128
EPT = 10112
EP = EPT * NTILES
NCHUNKS = EPT // CHUNK
ROWS_PER_TILE = NP // NSUB
TRASH = NN
ALPHA = 1.0 / (NLAYERS + 1)


def _sc_layer_body(cur_hbm, src_hbm, dst_hbm, w_hbm, zero_hbm, out_hbm,
                   sidx, didx, wbuf, rows, acc_sh, gsem):
    c = lax.axis_index("c")
    s = lax.axis_index("s")
    wid = s * NCORES + c
    r0 = s * ROWS_PER_TILE
    pltpu.sync_copy(zero_hbm.at[pl.ds(r0, ROWS_PER_TILE)],
                    acc_sh.at[pl.ds(r0, ROWS_PER_TILE)])
    plsc.subcore_barrier()

    ebase = wid * EPT

    def chunk(g, carry):
        e0 = ebase + g * CHUNK
        pltpu.sync_copy(src_hbm.at[pl.ds(e0, CHUNK)], sidx)
        pltpu.sync_copy(dst_hbm.at[pl.ds(e0, CHUNK)], didx)
        pltpu.sync_copy(w_hbm.at[pl.ds(e0, CHUNK)], wbuf)
        pltpu.async_copy(cur_hbm.at[sidx], rows, gsem).wait()

        def escale(g16, _):
            wv = wbuf[pl.ds(g16 * 16, 16)]
            for e in range(16):
                w = wv[e]
                row = g16 * 16 + e
                for cc in range(EMB // 16):
                    sl = pl.ds(cc * 16, 16)
                    rows[row, sl] = rows[row, sl] * w
            return 0

        lax.fori_loop(0, CHUNK // 16, escale, 0)
        pltpu.sync_copy(rows, acc_sh.at[didx], add=True)
        return carry

    lax.fori_loop(0, NCHUNKS, chunk, 0)
    plsc.subcore_barrier()
    pltpu.sync_copy(acc_sh.at[pl.ds(r0, ROWS_PER_TILE)],
                    out_hbm.at[c, pl.ds(r0, ROWS_PER_TILE)])


_sc_layer = pl.kernel(
    _sc_layer_body,
    out_type=jax.ShapeDtypeStruct((NCORES, NP, EMB), jnp.float32),
    mesh=plsc.VectorSubcoreMesh(core_axis_name="c", subcore_axis_name="s",
                                num_cores=NCORES, num_subcores=NSUB),
    scratch_types=[
        pltpu.VMEM((CHUNK,), jnp.int32),
        pltpu.VMEM((CHUNK,), jnp.int32),
        pltpu.VMEM((CHUNK,), jnp.float32),
        pltpu.VMEM((CHUNK, EMB), jnp.float32),
        pltpu.VMEM_SHARED((NP, EMB), jnp.float32),
        pltpu.SemaphoreType.DMA,
    ],
)

_BLK = 1280


def _combine_mid_body(p_ref, acc_ref, cur_out, acc_out):
    cur = p_ref[0] + p_ref[1]
    cur_out[...] = cur
    acc_out[...] = acc_ref[...] + cur


def _combine_mid(parts, acc):
    return pl.pallas_call(
        _combine_mid_body,
        grid=(NP // _BLK,),
        in_specs=[pl.BlockSpec((NCORES, _BLK, EMB), lambda i: (0, i, 0)),
                  pl.BlockSpec((_BLK, EMB), lambda i: (i, 0))],
        out_specs=[pl.BlockSpec((_BLK, EMB), lambda i: (i, 0)),
                   pl.BlockSpec((_BLK, EMB), lambda i: (i, 0))],
        out_shape=[jax.ShapeDtypeStruct((NP, EMB), jnp.float32),
                   jax.ShapeDtypeStruct((NP, EMB), jnp.float32)],
    )(parts, acc)


def _combine_last_body(p_ref, acc_ref, out_ref):
    out_ref[...] = ALPHA * (acc_ref[...] + p_ref[0] + p_ref[1])


def _combine_last(parts, acc):
    return pl.pallas_call(
        _combine_last_body,
        grid=(NP // _BLK,),
        in_specs=[pl.BlockSpec((NCORES, _BLK, EMB), lambda i: (0, i, 0)),
                  pl.BlockSpec((_BLK, EMB), lambda i: (i, 0))],
        out_specs=pl.BlockSpec((_BLK, EMB), lambda i: (i, 0)),
        out_shape=jax.ShapeDtypeStruct((NP, EMB), jnp.float32),
    )(parts, acc)


def kernel(edge_index, edge_weight, user_emb, item_emb):
    src = edge_index[0].astype(jnp.int32)
    dst = edge_index[1].astype(jnp.int32)
    w = edge_weight.astype(jnp.float32)
    pad_e = EP - E
    src = jnp.concatenate([src, jnp.zeros((pad_e,), jnp.int32)])
    dst = jnp.concatenate([dst, jnp.full((pad_e,), TRASH, jnp.int32)])
    w = jnp.concatenate([w, jnp.zeros((pad_e,), jnp.float32)])
    ego = jnp.concatenate([user_emb, item_emb], axis=0)
    cur = jnp.pad(ego, ((0, NP - NN), (0, 0)))
    zeros = jnp.zeros((NP, EMB), jnp.float32)
    acc = cur
    out = None
    for layer in range(NLAYERS):
        parts = _sc_layer(cur, src, dst, w, zeros)
        if layer < NLAYERS - 1:
            cur, acc = _combine_mid(parts, acc)
        else:
            out = _combine_last(parts, acc)
    return (out[:N_USERS], out[N_USERS:NN])


# speedup vs baseline: 4.9735x; 1.5392x over previous
import jax
import jax.numpy as jnp
from jax import lax
from jax.experimental import pallas as pl
from jax.experimental.pallas import tpu as pltpu
from jax.experimental.pallas import tpu_sc as plsc

N_USERS = 5000
N_ITEMS = 5000
NN = N_USERS + N_ITEMS
NP = 10112
EMB = 128
NLAYERS = 3
E = 320000
NCORES = 2
NSUB = 16
NTILES = NCORES * NSUB
CHUNK = 80
NCHUNKS = 126
EPT = CHUNK * NCHUNKS
EP = EPT * NTILES
ROWS_PER_TILE = NP // NSUB
TRASH = NN
ALPHA = 1.0 / (NLAYERS + 1)


def _sc_layer_body(cur_hbm, src_hbm, dst_hbm, w_hbm, zero_hbm, out_hbm,
                   sidx0, sidx1, didx0, didx1, didxs0, didxs1, wbuf0, wbuf1,
                   rows0, rows1, srow0, srow1, acc_sh,
                   gsem0, gsem1, ssem0, ssem1, esem):
    c = lax.axis_index("c")
    s = lax.axis_index("s")
    wid = s * NCORES + c
    r0 = s * ROWS_PER_TILE
    pltpu.sync_copy(zero_hbm.at[pl.ds(r0, ROWS_PER_TILE)],
                    acc_sh.at[pl.ds(r0, ROWS_PER_TILE)])
    plsc.subcore_barrier()

    sidx = (sidx0, sidx1)
    didx = (didx0, didx1)
    didxs = (didxs0, didxs1)
    wbuf = (wbuf0, wbuf1)
    rows = (rows0, rows1)
    srow = (srow0, srow1)
    gsem = (gsem0, gsem1)
    ssem = (ssem0, ssem1)

    def fire_idx(g, b):
        pltpu.async_copy(src_hbm.at[wid, g], sidx[b], esem)
        pltpu.async_copy(dst_hbm.at[wid, g], didx[b], esem)
        pltpu.async_copy(w_hbm.at[wid, g], wbuf[b], esem)

    def wait_idx(g, b):
        pltpu.make_async_copy(src_hbm.at[wid, g], sidx[b], esem).wait()
        pltpu.make_async_copy(dst_hbm.at[wid, g], didx[b], esem).wait()
        pltpu.make_async_copy(w_hbm.at[wid, g], wbuf[b], esem).wait()

    def fire_gather(g, b):
        pltpu.async_copy(cur_hbm.at[sidx[b]], rows[b], gsem[b])

    def wait_gather(b):
        pltpu.make_async_copy(cur_hbm.at[sidx[b]], rows[b], gsem[b]).wait()

    def fire_scatter(b):
        pltpu.async_copy(srow[b], acc_sh.at[didxs[b]], ssem[b], add=True)

    def wait_scatter(b):
        pltpu.make_async_copy(srow[b], acc_sh.at[didxs[b]], ssem[b]).wait()

    fire_idx(0, 0)
    fire_idx(1, 1)
    wait_idx(0, 0)
    fire_gather(0, 0)

    def step(g, b):
        o = 1 - b
        wait_gather(b)

        @pl.when(g + 1 < NCHUNKS)
        def _():
            wait_idx(g + 1, o)
            fire_gather(g + 1, o)

        @pl.when(g >= 2)
        def _():
            wait_scatter(b)

        for i in range(CHUNK // 16):
            didxs[b][pl.ds(i * 16, 16)] = didx[b][pl.ds(i * 16, 16)]

        def escale(g16, _):
            wv = wbuf[b][pl.ds(g16 * 16, 16)]
            for e in range(16):
                w = wv[e]
                row = g16 * 16 + e
                for cc in range(EMB // 16):
                    sl = pl.ds(cc * 16, 16)
                    srow[b][row, sl] = rows[b][row, sl] * w
            return 0

        lax.fori_loop(0, CHUNK // 16, escale, 0)
        fire_scatter(b)

        @pl.when(g + 2 < NCHUNKS)
        def _():
            fire_idx(g + 2, b)

    def outer(i, carry):
        step(i * 2, 0)
        step(i * 2 + 1, 1)
        return carry

    lax.fori_loop(0, NCHUNKS // 2, outer, 0)
    wait_scatter(0)
    wait_scatter(1)
    plsc.subcore_barrier()
    pltpu.sync_copy(acc_sh.at[pl.ds(r0, ROWS_PER_TILE)],
                    out_hbm.at[c, pl.ds(r0, ROWS_PER_TILE)])


_sc_layer = pl.kernel(
    _sc_layer_body,
    out_type=jax.ShapeDtypeStruct((NCORES, NP, EMB), jnp.float32),
    mesh=plsc.VectorSubcoreMesh(core_axis_name="c", subcore_axis_name="s",
                                num_cores=NCORES, num_subcores=NSUB),
    scratch_types=[
        pltpu.VMEM((CHUNK,), jnp.int32),
        pltpu.VMEM((CHUNK,), jnp.int32),
        pltpu.VMEM((CHUNK,), jnp.int32),
        pltpu.VMEM((CHUNK,), jnp.int32),
        pltpu.VMEM((CHUNK,), jnp.int32),
        pltpu.VMEM((CHUNK,), jnp.int32),
        pltpu.VMEM((CHUNK,), jnp.float32),
        pltpu.VMEM((CHUNK,), jnp.float32),
        pltpu.VMEM((CHUNK, EMB), jnp.float32),
        pltpu.VMEM((CHUNK, EMB), jnp.float32),
        pltpu.VMEM((CHUNK, EMB), jnp.float32),
        pltpu.VMEM((CHUNK, EMB), jnp.float32),
        pltpu.VMEM_SHARED((NP, EMB), jnp.float32),
        pltpu.SemaphoreType.DMA,
        pltpu.SemaphoreType.DMA,
        pltpu.SemaphoreType.DMA,
        pltpu.SemaphoreType.DMA,
        pltpu.SemaphoreType.DMA,
    ],
)

_BLK = 1264


def _combine_mid_body(p_ref, acc_ref, cur_out, acc_out):
    cur = p_ref[0] + p_ref[1]
    cur_out[...] = cur
    acc_out[...] = acc_ref[...] + cur


def _combine_mid(parts, acc):
    return pl.pallas_call(
        _combine_mid_body,
        grid=(NP // _BLK,),
        in_specs=[pl.BlockSpec((NCORES, _BLK, EMB), lambda i: (0, i, 0)),
                  pl.BlockSpec((_BLK, EMB), lambda i: (i, 0))],
        out_specs=[pl.BlockSpec((_BLK, EMB), lambda i: (i, 0)),
                   pl.BlockSpec((_BLK, EMB), lambda i: (i, 0))],
        out_shape=[jax.ShapeDtypeStruct((NP, EMB), jnp.float32),
                   jax.ShapeDtypeStruct((NP, EMB), jnp.float32)],
    )(parts, acc)


def _combine_last_body(p_ref, acc_ref, out_ref):
    out_ref[...] = ALPHA * (acc_ref[...] + p_ref[0] + p_ref[1])


def _combine_last(parts, acc):
    return pl.pallas_call(
        _combine_last_body,
        grid=(NP // _BLK,),
        in_specs=[pl.BlockSpec((NCORES, _BLK, EMB), lambda i: (0, i, 0)),
                  pl.BlockSpec((_BLK, EMB), lambda i: (i, 0))],
        out_specs=pl.BlockSpec((_BLK, EMB), lambda i: (i, 0)),
        out_shape=jax.ShapeDtypeStruct((NP, EMB), jnp.float32),
    )(parts, acc)


def kernel(edge_index, edge_weight, user_emb, item_emb):
    src = edge_index[0].astype(jnp.int32)
    dst = edge_index[1].astype(jnp.int32)
    w = edge_weight.astype(jnp.float32)
    pad_e = EP - E
    src = jnp.concatenate([src, jnp.zeros((pad_e,), jnp.int32)])
    dst = jnp.concatenate([dst, jnp.full((pad_e,), TRASH, jnp.int32)])
    w = jnp.concatenate([w, jnp.zeros((pad_e,), jnp.float32)])
    src = src.reshape(NTILES, NCHUNKS, CHUNK)
    dst = dst.reshape(NTILES, NCHUNKS, CHUNK)
    w = w.reshape(NTILES, NCHUNKS, CHUNK)
    ego = jnp.concatenate([user_emb, item_emb], axis=0)
    cur = jnp.pad(ego, ((0, NP - NN), (0, 0)))
    zeros = jnp.zeros((NP, EMB), jnp.float32)
    acc = cur
    out = None
    for layer in range(NLAYERS):
        parts = _sc_layer(cur, src, dst, w, zeros)
        if layer < NLAYERS - 1:
            cur, acc = _combine_mid(parts, acc)
        else:
            out = _combine_last(parts, acc)
    return (out[:N_USERS], out[N_USERS:NN])


# speedup vs baseline: 4.9740x; 1.0001x over previous
import jax
import jax.numpy as jnp
from jax import lax
from jax.experimental import pallas as pl
from jax.experimental.pallas import tpu as pltpu
from jax.experimental.pallas import tpu_sc as plsc

N_USERS = 5000
N_ITEMS = 5000
NN = N_USERS + N_ITEMS
NP = 10112
EMB = 128
NLAYERS = 3
E = 320000
NCORES = 2
NSUB = 16
NTILES = NCORES * NSUB
CHUNK = 80
NCHUNKS = 126
EPT = CHUNK * NCHUNKS
EP = EPT * NTILES
ROWS_PER_TILE = NP // NSUB
TRASH = NN
ALPHA = 1.0 / (NLAYERS + 1)


def _sc_layer_body(cur_hbm, edata_hbm, zero_hbm, out_hbm,
                   ebuf0, ebuf1, didxs0, didxs1,
                   rows0, rows1, srow0, srow1, acc_sh,
                   gsem0, gsem1, ssem0, ssem1, esem):
    c = lax.axis_index("c")
    s = lax.axis_index("s")
    wid = s * NCORES + c
    r0 = s * ROWS_PER_TILE
    pltpu.sync_copy(zero_hbm.at[pl.ds(r0, ROWS_PER_TILE)],
                    acc_sh.at[pl.ds(r0, ROWS_PER_TILE)])
    plsc.subcore_barrier()

    ebuf = (ebuf0, ebuf1)
    didxs = (didxs0, didxs1)
    rows = (rows0, rows1)
    srow = (srow0, srow1)
    gsem = (gsem0, gsem1)
    ssem = (ssem0, ssem1)

    def fire_idx(g, b):
        pltpu.async_copy(edata_hbm.at[wid, g], ebuf[b], esem)

    def wait_idx(g, b):
        pltpu.make_async_copy(edata_hbm.at[wid, g], ebuf[b], esem).wait()

    def fire_gather(g, b):
        pltpu.async_copy(cur_hbm.at[ebuf[b].at[0]], rows[b], gsem[b])

    def wait_gather(b):
        pltpu.make_async_copy(cur_hbm.at[ebuf[b].at[0]], rows[b],
                              gsem[b]).wait()

    def fire_scatter(b):
        pltpu.async_copy(srow[b], acc_sh.at[didxs[b]], ssem[b], add=True)

    def wait_scatter(b):
        pltpu.make_async_copy(srow[b], acc_sh.at[didxs[b]], ssem[b]).wait()

    fire_idx(0, 0)
    fire_idx(1, 1)
    wait_idx(0, 0)
    fire_gather(0, 0)

    def step(g, b):
        o = 1 - b
        wait_gather(b)

        @pl.when(g + 1 < NCHUNKS)
        def _():
            wait_idx(g + 1, o)
            fire_gather(g + 1, o)

        @pl.when(g >= 2)
        def _():
            wait_scatter(b)

        for i in range(CHUNK // 16):
            didxs[b][pl.ds(i * 16, 16)] = ebuf[b][1, pl.ds(i * 16, 16)]

        def escale(g16, _):
            wv = jax.lax.bitcast_convert_type(
                ebuf[b][2, pl.ds(g16 * 16, 16)], jnp.float32)
            for e in range(16):
                w = wv[e]
                row = g16 * 16 + e
                for cc in range(EMB // 16):
                    sl = pl.ds(cc * 16, 16)
                    srow[b][row, sl] = rows[b][row, sl] * w
            return 0

        lax.fori_loop(0, CHUNK // 16, escale, 0)
        fire_scatter(b)

        @pl.when(g + 2 < NCHUNKS)
        def _():
            fire_idx(g + 2, b)

    def outer(i, carry):
        step(i * 2, 0)
        step(i * 2 + 1, 1)
        return carry

    lax.fori_loop(0, NCHUNKS // 2, outer, 0)
    wait_scatter(0)
    wait_scatter(1)
    plsc.subcore_barrier()
    pltpu.sync_copy(acc_sh.at[pl.ds(r0, ROWS_PER_TILE)],
                    out_hbm.at[c, pl.ds(r0, ROWS_PER_TILE)])


_sc_layer = pl.kernel(
    _sc_layer_body,
    out_type=jax.ShapeDtypeStruct((NCORES, NP, EMB), jnp.float32),
    mesh=plsc.VectorSubcoreMesh(core_axis_name="c", subcore_axis_name="s",
                                num_cores=NCORES, num_subcores=NSUB),
    scratch_types=[
        pltpu.VMEM((3, CHUNK), jnp.int32),
        pltpu.VMEM((3, CHUNK), jnp.int32),
        pltpu.VMEM((CHUNK,), jnp.int32),
        pltpu.VMEM((CHUNK,), jnp.int32),
        pltpu.VMEM((CHUNK, EMB), jnp.float32),
        pltpu.VMEM((CHUNK, EMB), jnp.float32),
        pltpu.VMEM((CHUNK, EMB), jnp.float32),
        pltpu.VMEM((CHUNK, EMB), jnp.float32),
        pltpu.VMEM_SHARED((NP, EMB), jnp.float32),
        pltpu.SemaphoreType.DMA,
        pltpu.SemaphoreType.DMA,
        pltpu.SemaphoreType.DMA,
        pltpu.SemaphoreType.DMA,
        pltpu.SemaphoreType.DMA,
    ],
)

_BLK = 1264


def _combine_mid_body(p_ref, acc_ref, cur_out, acc_out):
    cur = p_ref[0] + p_ref[1]
    cur_out[...] = cur
    acc_out[...] = acc_ref[...] + cur


def _combine_mid(parts, acc):
    return pl.pallas_call(
        _combine_mid_body,
        grid=(NP // _BLK,),
        in_specs=[pl.BlockSpec((NCORES, _BLK, EMB), lambda i: (0, i, 0)),
                  pl.BlockSpec((_BLK, EMB), lambda i: (i, 0))],
        out_specs=[pl.BlockSpec((_BLK, EMB), lambda i: (i, 0)),
                   pl.BlockSpec((_BLK, EMB), lambda i: (i, 0))],
        out_shape=[jax.ShapeDtypeStruct((NP, EMB), jnp.float32),
                   jax.ShapeDtypeStruct((NP, EMB), jnp.float32)],
    )(parts, acc)


def _combine_last_body(p_ref, acc_ref, out_ref):
    out_ref[...] = ALPHA * (acc_ref[...] + p_ref[0] + p_ref[1])


def _combine_last(parts, acc):
    return pl.pallas_call(
        _combine_last_body,
        grid=(NP // _BLK,),
        in_specs=[pl.BlockSpec((NCORES, _BLK, EMB), lambda i: (0, i, 0)),
                  pl.BlockSpec((_BLK, EMB), lambda i: (i, 0))],
        out_specs=pl.BlockSpec((_BLK, EMB), lambda i: (i, 0)),
        out_shape=jax.ShapeDtypeStruct((NP, EMB), jnp.float32),
    )(parts, acc)


def kernel(edge_index, edge_weight, user_emb, item_emb):
    src = edge_index[0].astype(jnp.int32)
    dst = edge_index[1].astype(jnp.int32)
    w = edge_weight.astype(jnp.float32)
    pad_e = EP - E
    src = jnp.concatenate([src, jnp.zeros((pad_e,), jnp.int32)])
    dst = jnp.concatenate([dst, jnp.full((pad_e,), TRASH, jnp.int32)])
    w = jnp.concatenate([w, jnp.zeros((pad_e,), jnp.float32)])
    w_bits = jax.lax.bitcast_convert_type(w, jnp.int32)
    edata = jnp.stack([src.reshape(NTILES, NCHUNKS, CHUNK),
                       dst.reshape(NTILES, NCHUNKS, CHUNK),
                       w_bits.reshape(NTILES, NCHUNKS, CHUNK)], axis=2)
    ego = jnp.concatenate([user_emb, item_emb], axis=0)
    cur = jnp.pad(ego, ((0, NP - NN), (0, 0)))
    zeros = jnp.zeros((NP, EMB), jnp.float32)
    acc = cur
    out = None
    for layer in range(NLAYERS):
        parts = _sc_layer(cur, edata, zeros)
        if layer < NLAYERS - 1:
            cur, acc = _combine_mid(parts, acc)
        else:
            out = _combine_last(parts, acc)
    return (out[:N_USERS], out[N_USERS:NN])
